# baseline (speedup 1.0000x reference)
import functools

import jax
import jax.numpy as jnp
from jax import lax
from jax.experimental import pallas as pl
from jax.experimental.pallas import tpu as pltpu
from jax.experimental.pallas import tpu_sc as plsc

_NC = 2
_NS = 16
_NW = _NC * _NS


def _sc_gather(users, pos_items, neg_items, gu2, gi2, tu2, beta128):
    B = users.shape[0]
    bw = B // _NW
    mesh = plsc.VectorSubcoreMesh(core_axis_name="c", subcore_axis_name="s")

    @functools.partial(
        pl.kernel,
        out_type=[
            jax.ShapeDtypeStruct((B, 128), jnp.float32),
            jax.ShapeDtypeStruct((B, 128), jnp.float32),
            jax.ShapeDtypeStruct((B, 128), jnp.float32),
            jax.ShapeDtypeStruct((B, 128), jnp.float32),
            jax.ShapeDtypeStruct((B,), jnp.float32),
            jax.ShapeDtypeStruct((B,), jnp.float32),
        ],
        mesh=mesh,
        compiler_params=pltpu.CompilerParams(needs_layout_passes=False),
        scratch_types=[
            pltpu.VMEM((bw,), jnp.int32),
            pltpu.VMEM((bw,), jnp.int32),
            pltpu.VMEM((bw,), jnp.int32),
            pltpu.VMEM((bw,), jnp.int32),
            pltpu.VMEM((bw,), jnp.int32),
            pltpu.VMEM((bw,), jnp.int32),
            pltpu.VMEM((bw,), jnp.int32),
            pltpu.VMEM((bw, 128), jnp.float32),
            pltpu.VMEM((bw, 128), jnp.float32),
            pltpu.VMEM((bw, 128), jnp.float32),
            pltpu.VMEM((bw, 128), jnp.float32),
            pltpu.VMEM((bw, 128), jnp.float32),
            pltpu.VMEM((bw, 128), jnp.float32),
            pltpu.VMEM((bw,), jnp.float32),
            pltpu.VMEM((bw,), jnp.float32),
            pltpu.SemaphoreType.DMA,
        ],
    )
    def k(users_h, pos_h, neg_h, gu_h, gi_h, tu_h, bi_h,
          ug_o, ut_o, gp_o, gn_o, bp_o, bn_o,
          uidx, pidx, nidx, uhalf, phalf, nhalf, bidx,
          ug_v, ut_v, gp_v, gn_v, bp16_v, bn16_v, bp_v, bn_v, sem):
        wid = lax.axis_index("s") * _NC + lax.axis_index("c")
        base = wid * bw
        pltpu.sync_copy(users_h.at[pl.ds(base, bw)], uidx)
        pltpu.sync_copy(pos_h.at[pl.ds(base, bw)], pidx)
        pltpu.sync_copy(neg_h.at[pl.ds(base, bw)], nidx)
        for q in range(bw // 16):
            sl = pl.ds(q * 16, 16)
            uhalf[sl] = jnp.right_shift(uidx[sl], 1)
            phalf[sl] = jnp.right_shift(pidx[sl], 1)
            nhalf[sl] = jnp.right_shift(nidx[sl], 1)
            bidx[sl] = jnp.right_shift(pidx[sl], 7)
        c0 = pltpu.async_copy(gu_h.at[uhalf], ug_v, sem)
        c1 = pltpu.async_copy(tu_h.at[uhalf], ut_v, sem)
        c2 = pltpu.async_copy(gi_h.at[phalf], gp_v, sem)
        c3 = pltpu.async_copy(gi_h.at[nhalf], gn_v, sem)
        c4 = pltpu.async_copy(bi_h.at[bidx], bp16_v, sem)
        c0.wait()
        c1.wait()
        c2.wait()
        c3.wait()
        c4.wait()
        for q in range(bw // 16):
            sl = pl.ds(q * 16, 16)
            bidx[sl] = jnp.right_shift(nidx[sl], 7)
        c5 = pltpu.async_copy(bi_h.at[bidx], bn16_v, sem)
        c5.wait()
        for q in range(bw // 16):
            sl = pl.ds(q * 16, 16)
            rows = lax.iota(jnp.int32, 16) + q * 16
            pcols = jnp.bitwise_and(pidx[sl], 127)
            ncols = jnp.bitwise_and(nidx[sl], 127)
            bp_v[sl] = plsc.load_gather(bp16_v, [rows, pcols])
            bn_v[sl] = plsc.load_gather(bn16_v, [rows, ncols])
        pltpu.sync_copy(ug_v, ug_o.at[pl.ds(base, bw)])
        pltpu.sync_copy(ut_v, ut_o.at[pl.ds(base, bw)])
        pltpu.sync_copy(gp_v, gp_o.at[pl.ds(base, bw)])
        pltpu.sync_copy(gn_v, gn_o.at[pl.ds(base, bw)])
        pltpu.sync_copy(bp_v, bp_o.at[pl.ds(base, bw)])
        pltpu.sync_copy(bn_v, bn_o.at[pl.ds(base, bw)])

    return k(users, pos_items, neg_items, gu2, gi2, tu2, beta128)


def _tc_phase1(pos_f, neg_f, E, beta_prime, users, pos_items, neg_items,
               ug2, ut2, gp2, gn2, bp, bn):
    B, F = pos_f.shape
    G = E.shape[1]
    RB = 512

    def _half(pair_ref, idx_ref):
        odd = jnp.bitwise_and(idx_ref[...], 1) == 1
        left = pair_ref[:, :G]
        right = pair_ref[:, G:]
        return jnp.where(odd, right, left)

    def body(pf, nf, e_r, bpr, u_r, p_r, n_r,
             ug_r, ut_r, gp_r, gn_r, bp_r, bn_r,
             s_o, t_o, ug_o, ut_o, gp_o, gn_o):
        ug = _half(ug_r, u_r)
        ut = _half(ut_r, u_r)
        gp = _half(gp_r, p_r)
        gn = _half(gn_r, n_r)
        ug_o[...] = ug
        ut_o[...] = ut
        gp_o[...] = gp
        gn_o[...] = gn
        fd = pf[...] - nf[...]
        tid = lax.dot_general(fd, e_r[...], (((1,), (0,)), ((), ())),
                              precision=lax.Precision.HIGHEST,
                              preferred_element_type=jnp.float32)
        tv = lax.dot_general(fd, bpr[...], (((1,), (0,)), ((), ())),
                             precision=lax.Precision.HIGHEST,
                             preferred_element_type=jnp.float32)
        ugdot = jnp.sum(ug * (gp - gn), axis=1, keepdims=True)
        utdot = jnp.sum(ut * tid, axis=1, keepdims=True)
        s_o[...] = (bp_r[...] - bn_r[...]) + ugdot + utdot
        t_o[...] = tv

    return pl.pallas_call(
        body,
        grid=(B // RB,),
        in_specs=[
            pl.BlockSpec((RB, F), lambda i: (i, 0)),
            pl.BlockSpec((RB, F), lambda i: (i, 0)),
            pl.BlockSpec((F, G), lambda i: (0, 0)),
            pl.BlockSpec((F, 1), lambda i: (0, 0)),
            pl.BlockSpec((RB, 1), lambda i: (i, 0)),
            pl.BlockSpec((RB, 1), lambda i: (i, 0)),
            pl.BlockSpec((RB, 1), lambda i: (i, 0)),
            pl.BlockSpec((RB, 128), lambda i: (i, 0)),
            pl.BlockSpec((RB, 128), lambda i: (i, 0)),
            pl.BlockSpec((RB, 128), lambda i: (i, 0)),
            pl.BlockSpec((RB, 128), lambda i: (i, 0)),
            pl.BlockSpec((RB, 1), lambda i: (i, 0)),
            pl.BlockSpec((RB, 1), lambda i: (i, 0)),
        ],
        out_specs=[
            pl.BlockSpec((RB, 1), lambda i: (i, 0)),
            pl.BlockSpec((RB, 1), lambda i: (i, 0)),
            pl.BlockSpec((RB, G), lambda i: (i, 0)),
            pl.BlockSpec((RB, G), lambda i: (i, 0)),
            pl.BlockSpec((RB, G), lambda i: (i, 0)),
            pl.BlockSpec((RB, G), lambda i: (i, 0)),
        ],
        out_shape=[
            jax.ShapeDtypeStruct((B, 1), jnp.float32),
            jax.ShapeDtypeStruct((B, 1), jnp.float32),
            jax.ShapeDtypeStruct((B, G), jnp.float32),
            jax.ShapeDtypeStruct((B, G), jnp.float32),
            jax.ShapeDtypeStruct((B, G), jnp.float32),
            jax.ShapeDtypeStruct((B, G), jnp.float32),
        ],
    )(pos_f, neg_f, E, beta_prime, users, pos_items, neg_items,
      ug2, ut2, gp2, gn2, bp, bn)


def _tc_fill(t, s_row):
    B = t.shape[0]
    RB = 512

    def body(t_r, s_r, out_r):
        out_r[...] = t_r[...] + s_r[...]

    return pl.pallas_call(
        body,
        grid=(B // RB,),
        in_specs=[
            pl.BlockSpec((RB, 1), lambda i: (i, 0)),
            pl.BlockSpec((1, B), lambda i: (0, 0)),
        ],
        out_specs=pl.BlockSpec((RB, B), lambda i: (i, 0)),
        out_shape=jax.ShapeDtypeStruct((B, B), jnp.float32),
    )(t, s_row)


def kernel(users, pos_items, neg_items, pos_items_features,
           neg_items_features, gamma_users, gamma_items, theta_users, E,
           beta_items, beta_prime):
    users = users.astype(jnp.int32)
    pos_items = pos_items.astype(jnp.int32)
    neg_items = neg_items.astype(jnp.int32)
    n_items = beta_items.shape[0]
    gu2 = jnp.reshape(gamma_users, (-1, 128))
    tu2 = jnp.reshape(theta_users, (-1, 128))
    gi2 = jnp.reshape(gamma_items, (-1, 128))
    beta_flat = jnp.reshape(beta_items, (n_items,))
    pad = (-n_items) % 128
    if pad:
        beta_flat = jnp.concatenate(
            [beta_flat, jnp.zeros((pad,), jnp.float32)])
    beta128 = jnp.reshape(beta_flat, (-1, 128))
    ug2, ut2, gp2, gn2, bp, bn = _sc_gather(
        users, pos_items, neg_items, gu2, gi2, tu2, beta128)
    bp = jnp.reshape(bp, (bp.shape[0], 1))
    bn = jnp.reshape(bn, (bn.shape[0], 1))
    s, t, ug, ut, gp, gn = _tc_phase1(
        pos_items_features, neg_items_features, E, beta_prime,
        jnp.reshape(users, (-1, 1)), jnp.reshape(pos_items, (-1, 1)),
        jnp.reshape(neg_items, (-1, 1)), ug2, ut2, gp2, gn2, bp, bn)
    Xuij = _tc_fill(t, jnp.transpose(s))
    return (Xuij, (ug, ut), (bp, bn), (gp, gn))
